# baseline (speedup 1.0000x reference)
import functools
import math

import jax
import jax.numpy as jnp
from jax import lax
from jax.experimental import pallas as pl
from jax.experimental.pallas import tpu as pltpu
from jax.experimental.pallas import tpu_sc as plsc

_PADDING_IDX = 0
_SMOOTHING = 0.1
_CONFIDENCE = 1.0 - _SMOOTHING

_NC = 2
_NS = 16
_NW = _NC * _NS
_L = 16


def _sc_gather_body(n, v, eps, row_const, x_hbm, t_hbm, out_hbm, t_v, gi_v,
                    zi_v, gv_v, zv_v, acc_v, sem_t, sem_g, sem_z):
    rows_per_w = n // _NW
    chunks = rows_per_w // _L
    wid = lax.axis_index("s") * _NC + lax.axis_index("c")
    base = wid * rows_per_w

    pltpu.async_copy(t_hbm.at[pl.ds(base, rows_per_w)], t_v, sem_t).wait()

    for c in range(chunks):
        tv = t_v[pl.ds(c * _L, _L)]
        nv = (base + c * _L + lax.iota(jnp.int32, _L)) * v
        gi_v[pl.ds(c * _L, _L)] = nv * 0 + (c * _L + lax.iota(jnp.int32, _L))
        zi_v[pl.ds(c * _L, _L)] = nv * 0 + (c * _L + lax.iota(jnp.int32, _L))

    dg = pltpu.async_copy(x_hbm.at[gi_v], gv_v, sem_g)
    dz = pltpu.async_copy(x_hbm.at[zi_v], zv_v, sem_z)
    dg.wait()
    dz.wait()

    acc = jnp.zeros((_L,), jnp.float32)
    for c in range(chunks):
        g = gv_v[pl.ds(c * _L, _L)]
        x0 = zv_v[pl.ds(c * _L, _L)]
        tv = t_v[pl.ds(c * _L, _L)]
        m = jnp.where(tv != _PADDING_IDX, 1.0, 0.0).astype(jnp.float32)
        acc = acc + m * ((eps - _CONFIDENCE) * g + eps * x0 + row_const)

    acc_v[...] = acc
    pltpu.sync_copy(acc_v, out_hbm.at[wid])


def _make_sc_gather(n, v, eps, row_const):
    rows_per_w = n // _NW
    body = functools.partial(_sc_gather_body, n, v, eps, row_const)
    return pl.kernel(
        body,
        mesh=plsc.VectorSubcoreMesh(core_axis_name="c", subcore_axis_name="s"),
        out_type=jax.ShapeDtypeStruct((_NW, _L), jnp.float32),
        scratch_types=[
            pltpu.VMEM((rows_per_w,), jnp.int32),
            pltpu.VMEM((rows_per_w,), jnp.int32),
            pltpu.VMEM((rows_per_w,), jnp.int32),
            pltpu.VMEM((rows_per_w,), jnp.float32),
            pltpu.VMEM((rows_per_w,), jnp.float32),
            pltpu.VMEM((_L,), jnp.float32),
            pltpu.SemaphoreType.DMA,
            pltpu.SemaphoreType.DMA,
            pltpu.SemaphoreType.DMA,
        ],
    )


def _red_kernel(t_ref, g_ref, x_ref, o_ref, *, eps):
    first = (pl.program_id(0) == 0) & (pl.program_id(1) == 0)

    x = x_ref[...]
    t = t_ref[0, 0, :]
    m = (t != _PADDING_IDX).astype(jnp.float32)
    bs = jnp.sum(x, axis=1)
    partial = -eps * jnp.sum(bs * m)

    @pl.when(first)
    def _():
        o_ref[...] = jnp.full((1, 1), jnp.sum(g_ref[...]), jnp.float32)

    o_ref[...] += jnp.full((1, 1), partial, dtype=jnp.float32)


def kernel(x, target):
    n, v = x.shape
    row_block = 512
    col_block = 3200
    nr = n // row_block
    nc = v // col_block

    eps = _SMOOTHING / (v - 2)
    row_const = _SMOOTHING * math.log(eps) + _CONFIDENCE * math.log(_CONFIDENCE)

    t32 = target.astype(jnp.int32)
    xflat = x.reshape(n * v)
    g = _make_sc_gather(n, v, eps, row_const)(xflat, t32)

    t3 = t32.reshape(nr, 1, row_block)

    out = pl.pallas_call(
        functools.partial(_red_kernel, eps=eps),
        grid=(nr, nc),
        in_specs=[
            pl.BlockSpec((1, 1, row_block), lambda i, j: (i, 0, 0)),
            pl.BlockSpec((_NW, _L), lambda i, j: (0, 0)),
            pl.BlockSpec((row_block, col_block), lambda i, j: (i, j)),
        ],
        out_specs=pl.BlockSpec((1, 1), lambda i, j: (0, 0)),
        out_shape=jax.ShapeDtypeStruct((1, 1), jnp.float32),
    )(t3, g, x)
    return out[0, 0]


# speedup vs baseline: 1.0600x; 1.0600x over previous
import functools
import math

import jax
import jax.numpy as jnp
from jax import lax
from jax.experimental import pallas as pl
from jax.experimental.pallas import tpu as pltpu
from jax.experimental.pallas import tpu_sc as plsc

_PADDING_IDX = 0
_SMOOTHING = 0.1
_CONFIDENCE = 1.0 - _SMOOTHING

_NC = 2
_NS = 16
_NW = _NC * _NS
_L = 16


def _sc_gather_body(n, v, x2_hbm, t_hbm, out_hbm, t_v, ridx_v, xg_v,
                    sem_t, sem_g):
    rows_per_w = n // _NW
    chunks = rows_per_w // _L
    tiles_per_row = v // 128
    wid = lax.axis_index("s") * _NC + lax.axis_index("c")
    base = wid * rows_per_w

    pltpu.async_copy(t_hbm.at[pl.ds(base, rows_per_w)], t_v, sem_t).wait()

    iota = lax.iota(jnp.int32, _L)
    for c in range(chunks):
        tv = t_v[pl.ds(c * _L, _L)]
        nv = (base + c * _L + iota) * tiles_per_row
        ridx_v[pl.ds(c * _L, _L)] = nv + lax.shift_right_logical(tv, 7)

    pltpu.async_copy(x2_hbm.at[ridx_v], xg_v, sem_g).wait()
    pltpu.sync_copy(xg_v, out_hbm.at[pl.ds(base, rows_per_w)])


def _make_sc_gather(n, v):
    rows_per_w = n // _NW
    body = functools.partial(_sc_gather_body, n, v)
    return pl.kernel(
        body,
        mesh=plsc.VectorSubcoreMesh(core_axis_name="c", subcore_axis_name="s"),
        out_type=jax.ShapeDtypeStruct((n, 128), jnp.float32),
        scratch_types=[
            pltpu.VMEM((rows_per_w,), jnp.int32),
            pltpu.VMEM((rows_per_w,), jnp.int32),
            pltpu.VMEM((rows_per_w, 128), jnp.float32),
            pltpu.SemaphoreType.DMA,
            pltpu.SemaphoreType.DMA,
        ],
    )


def _red_kernel(t_ref, g_ref, x_ref, o_ref, *, eps, row_const):
    j = pl.program_id(1)
    first = (pl.program_id(0) == 0) & (j == 0)

    x = x_ref[...]
    t = t_ref[0, 0, :]
    m = (t != _PADDING_IDX).astype(jnp.float32)
    partial = -eps * jnp.sum(jnp.sum(x, axis=1) * m)

    @pl.when(first)
    def _():
        o_ref[...] = jnp.zeros_like(o_ref)

    @pl.when(j == 0)
    def _():
        gblk = g_ref[...]
        lane = (t & 127)[:, None]
        li = jax.lax.broadcasted_iota(jnp.int32, gblk.shape, 1)
        gsel = jnp.sum(jnp.where(li == lane, gblk, 0.0), axis=1)
        head = (
            (eps - _CONFIDENCE) * jnp.sum(gsel * m)
            + eps * jnp.sum(x[:, 0] * m)
            + row_const * jnp.sum(m)
        )
        o_ref[...] += jnp.full((1, 1), head, dtype=jnp.float32)

    o_ref[...] += jnp.full((1, 1), partial, dtype=jnp.float32)


def kernel(x, target):
    n, v = x.shape
    row_block = 512
    col_block = 3200
    nr = n // row_block
    nc = v // col_block

    eps = _SMOOTHING / (v - 2)
    row_const = _SMOOTHING * math.log(eps) + _CONFIDENCE * math.log(_CONFIDENCE)

    t32 = target.astype(jnp.int32)
    x2 = x.reshape(n * v // 128, 128)
    g = _make_sc_gather(n, v)(x2, t32)

    t3 = t32.reshape(nr, 1, row_block)

    out = pl.pallas_call(
        functools.partial(_red_kernel, eps=eps, row_const=row_const),
        grid=(nr, nc),
        in_specs=[
            pl.BlockSpec((1, 1, row_block), lambda i, j: (i, 0, 0)),
            pl.BlockSpec((row_block, 128), lambda i, j: (i, 0)),
            pl.BlockSpec((row_block, col_block), lambda i, j: (i, j)),
        ],
        out_specs=pl.BlockSpec((1, 1), lambda i, j: (0, 0)),
        out_shape=jax.ShapeDtypeStruct((1, 1), jnp.float32),
    )(t3, g, x)
    return out[0, 0]


# speedup vs baseline: 3.0690x; 2.8953x over previous
import functools
import math

import jax
import jax.numpy as jnp
from jax.experimental import pallas as pl

_PADDING_IDX = 0
_SMOOTHING = 0.1
_CONFIDENCE = 1.0 - _SMOOTHING


def _loss_kernel(t_ref, x_ref, o_ref, *, col_block, eps, row_const):
    j = pl.program_id(1)
    first = (pl.program_id(0) == 0) & (j == 0)

    x = x_ref[...]
    t = t_ref[0, 0, :]
    m = (t != _PADDING_IDX).astype(jnp.float32)

    cols = j * col_block + jax.lax.broadcasted_iota(jnp.int32, x.shape, 1)
    sel = jnp.where(cols == t[:, None], x, 0.0)
    s = jnp.sum(x, axis=1)
    gs = jnp.sum(sel, axis=1)
    partial = jnp.sum(m * ((eps - _CONFIDENCE) * gs - eps * s))

    @pl.when(first)
    def _():
        o_ref[...] = jnp.zeros_like(o_ref)

    @pl.when(j == 0)
    def _():
        head = eps * jnp.sum(x[:, 0] * m) + row_const * jnp.sum(m)
        o_ref[...] += jnp.full((1, 1), head, dtype=jnp.float32)

    o_ref[...] += jnp.full((1, 1), partial, dtype=jnp.float32)


def kernel(x, target):
    n, v = x.shape
    row_block = 512
    col_block = 3200
    nr = n // row_block
    nc = v // col_block

    eps = _SMOOTHING / (v - 2)
    row_const = _SMOOTHING * math.log(eps) + _CONFIDENCE * math.log(_CONFIDENCE)

    t32 = target.astype(jnp.int32)
    t3 = t32.reshape(nr, 1, row_block)

    out = pl.pallas_call(
        functools.partial(
            _loss_kernel, col_block=col_block, eps=eps, row_const=row_const
        ),
        grid=(nr, nc),
        in_specs=[
            pl.BlockSpec((1, 1, row_block), lambda i, j: (i, 0, 0)),
            pl.BlockSpec((row_block, col_block), lambda i, j: (i, j)),
        ],
        out_specs=pl.BlockSpec((1, 1), lambda i, j: (0, 0)),
        out_shape=jax.ShapeDtypeStruct((1, 1), jnp.float32),
    )(t3, x)
    return out[0, 0]


# speedup vs baseline: 3.5848x; 1.1681x over previous
import functools
import math

import jax
import jax.numpy as jnp
from jax.experimental import pallas as pl

_PADDING_IDX = 0
_SMOOTHING = 0.1
_CONFIDENCE = 1.0 - _SMOOTHING


def _loss_kernel(t_ref, x_ref, o_ref, *, col_block, eps, row_const):
    j = pl.program_id(1)
    first = (pl.program_id(0) == 0) & (j == 0)

    x = x_ref[...]
    t = t_ref[0, 0, :]
    m = (t != _PADDING_IDX).astype(jnp.float32)

    cols = j * col_block + jax.lax.broadcasted_iota(jnp.int32, x.shape, 1)
    sel = jnp.where(cols == t[:, None], x, 0.0)
    s = jnp.sum(x, axis=1)
    gs = jnp.sum(sel, axis=1)
    partial = jnp.sum(m * ((eps - _CONFIDENCE) * gs - eps * s))

    @pl.when(first)
    def _():
        o_ref[...] = jnp.zeros_like(o_ref)

    @pl.when(j == 0)
    def _():
        head = eps * jnp.sum(x[:, 0] * m) + row_const * jnp.sum(m)
        o_ref[...] += jnp.full((1, 1), head, dtype=jnp.float32)

    o_ref[...] += jnp.full((1, 1), partial, dtype=jnp.float32)


def kernel(x, target):
    n, v = x.shape
    row_block = 512
    col_block = 6400
    nr = n // row_block
    nc = v // col_block

    eps = _SMOOTHING / (v - 2)
    row_const = _SMOOTHING * math.log(eps) + _CONFIDENCE * math.log(_CONFIDENCE)

    t32 = target.astype(jnp.int32)
    t3 = t32.reshape(nr, 1, row_block)

    out = pl.pallas_call(
        functools.partial(
            _loss_kernel, col_block=col_block, eps=eps, row_const=row_const
        ),
        grid=(nr, nc),
        in_specs=[
            pl.BlockSpec((1, 1, row_block), lambda i, j: (i, 0, 0)),
            pl.BlockSpec((row_block, col_block), lambda i, j: (i, j)),
        ],
        out_specs=pl.BlockSpec((1, 1), lambda i, j: (0, 0)),
        out_shape=jax.ShapeDtypeStruct((1, 1), jnp.float32),
    )(t3, x)
    return out[0, 0]


# speedup vs baseline: 3.6621x; 1.0216x over previous
import functools
import math

import jax
import jax.numpy as jnp
from jax.experimental import pallas as pl

_PADDING_IDX = 0
_SMOOTHING = 0.1
_CONFIDENCE = 1.0 - _SMOOTHING


def _loss_kernel(t_ref, x_ref, o_ref, *, col_block, eps, row_const):
    j = pl.program_id(1)
    first = (pl.program_id(0) == 0) & (j == 0)

    x = x_ref[...]
    t = t_ref[0, 0, :]
    m = (t != _PADDING_IDX).astype(jnp.float32)

    cols = j * col_block + jax.lax.broadcasted_iota(jnp.int32, x.shape, 1)
    sel = jnp.where(cols == t[:, None], x, 0.0)
    s = jnp.sum(x, axis=1)
    gs = jnp.sum(sel, axis=1)
    partial = jnp.sum(m * ((eps - _CONFIDENCE) * gs - eps * s))

    @pl.when(first)
    def _():
        o_ref[...] = jnp.zeros_like(o_ref)

    @pl.when(j == 0)
    def _():
        head = eps * jnp.sum(x[:, 0] * m) + row_const * jnp.sum(m)
        o_ref[...] += jnp.full((1, 1), head, dtype=jnp.float32)

    o_ref[...] += jnp.full((1, 1), partial, dtype=jnp.float32)


def kernel(x, target):
    n, v = x.shape
    row_block = 256
    col_block = 16000
    nr = n // row_block
    nc = v // col_block

    eps = _SMOOTHING / (v - 2)
    row_const = _SMOOTHING * math.log(eps) + _CONFIDENCE * math.log(_CONFIDENCE)

    t32 = target.astype(jnp.int32)
    t3 = t32.reshape(nr, 1, row_block)

    out = pl.pallas_call(
        functools.partial(
            _loss_kernel, col_block=col_block, eps=eps, row_const=row_const
        ),
        grid=(nr, nc),
        in_specs=[
            pl.BlockSpec((1, 1, row_block), lambda i, j: (i, 0, 0)),
            pl.BlockSpec((row_block, col_block), lambda i, j: (i, j)),
        ],
        out_specs=pl.BlockSpec((1, 1), lambda i, j: (0, 0)),
        out_shape=jax.ShapeDtypeStruct((1, 1), jnp.float32),
    )(t3, x)
    return out[0, 0]
